# baseline (speedup 1.0000x reference)
import functools

import jax
import jax.numpy as jnp
from jax import lax
from jax.experimental import pallas as pl
from jax.experimental.pallas import tpu as pltpu
from jax.experimental.pallas import tpu_sc as plsc

BATCH = 16384
DIM = 32
LANES = 16

_info = plsc.get_sparse_core_info()
NC = _info.num_cores
NS = _info.num_subcores
NW = NC * NS
NUM_ROWS = 1000000
CHUNK = 512
CPW = 61
RPW = CPW * CHUNK

_mesh = plsc.VectorSubcoreMesh(core_axis_name="c", subcore_axis_name="s")


@functools.partial(
    pl.kernel,
    mesh=_mesh,
    compiler_params=pltpu.CompilerParams(
        needs_layout_passes=False, use_tc_tiling_on_sc=True),
    out_type=(jax.ShapeDtypeStruct((BATCH, DIM), jnp.float32),
              jax.ShapeDtypeStruct((BATCH, DIM), jnp.float32)),
    scratch_types=[
        pltpu.VMEM((BATCH,), jnp.int32),
        pltpu.VMEM((BATCH,), jnp.int32),
        pltpu.VMEM((BATCH,), jnp.int32),
        pltpu.VMEM((BATCH,), jnp.int32),
        pltpu.VMEM((DIM, CHUNK), jnp.float32),
        pltpu.VMEM((DIM, CHUNK), jnp.float32),
        pltpu.VMEM((DIM, 64), jnp.float32),
        pltpu.VMEM((2 * LANES,), jnp.int32),
        pltpu.VMEM((2 * LANES,), jnp.int32),
        pltpu.VMEM((2 * LANES,), jnp.int32),
        pltpu.VMEM((2 * LANES,), jnp.int32),
        pltpu.VMEM((LANES, DIM), jnp.float32),
        pltpu.VMEM((LANES, DIM), jnp.float32),
        pltpu.SemaphoreType.DMA,
        pltpu.SemaphoreType.DMA,
        pltpu.SemaphoreType.DMA,
    ],
)
def _gather_stream(table_t_hbm, tail_t_hbm, ids1_hbm, ids2_hbm, g1_hbm,
                   g2_hbm,
                   idsb1, idsb2, posb1, posb2, buf0, buf1, tbuf,
                   pp1, pc1, pp2, pc2, stag1, stag2,
                   semb0, semb1, seme):
    wid = lax.axis_index("s") * NC + lax.axis_index("c")
    lo = wid * RPW
    nch = jnp.where(wid == NW - 1, CPW + 1, CPW)

    lanesv = lax.iota(jnp.int32, LANES)

    pltpu.sync_copy(ids1_hbm, idsb1)
    pltpu.sync_copy(ids2_hbm, idsb2)

    hi = jnp.where(wid == NW - 1, NUM_ROWS, lo + RPW)

    def scan_side(idsb, posb):
        def body(i, cnt):
            idv = idsb[pl.ds(i * LANES, LANES)]
            m = (idv >= lo) & (idv < hi)
            plsc.store_compressed(posb.at[pl.ds(cnt, LANES)],
                                  i * LANES + lanesv, mask=m)
            return cnt + plsc.all_reduce_population_count(m)[0]
        return lax.fori_loop(0, BATCH // LANES, body, jnp.int32(0))

    cnt1 = scan_side(idsb1, posb1)
    cnt2 = scan_side(idsb2, posb2)

    def fire(q, buf, semb):
        off = pl.multiple_of(lo + q * CHUNK, 128)
        pltpu.async_copy(table_t_hbm.at[:, pl.ds(off, CHUNK)], buf, semb)

    def bwait(buf, semb):
        pltpu.make_async_copy(
            table_t_hbm.at[:, pl.ds(0, CHUNK)], buf, semb).wait()

    def emit(buf, width, k, pp, pc, stag, g_hbm):
        posv = pp[pl.ds(0, LANES)]
        colv = pc[pl.ds(0, LANES)]
        emask = lanesv < k
        colv = jnp.where(emask, colv, 0)
        for c in range(DIM):
            cc = jnp.full((LANES,), c, jnp.int32)
            vals = plsc.load_gather(buf, [cc, colv], mask=emask)
            plsc.store_scatter(stag, [lanesv, cc], vals, mask=emask)
        for s in range(LANES):
            @pl.when(s < k)
            def _():
                ps = posv[s]
                pltpu.async_copy(stag.at[pl.ds(s, 1)],
                                 g_hbm.at[pl.ds(ps, 1)], seme)
        def drain(s, carry):
            pltpu.make_async_copy(stag.at[pl.ds(0, 1)],
                                  g_hbm.at[pl.ds(0, 1)], seme).wait()
            return carry
        lax.fori_loop(0, k, drain, 0)

    def process_side(buf, width, chunk_lo, cnt, idsb, posb, pp, pc, stag,
                     g_hbm):
        nvec = (cnt + LANES - 1) // LANES

        def body(j, pcnt):
            posv = posb[pl.ds(j * LANES, LANES)]
            vmask = (j * LANES + lanesv) < cnt
            posv = jnp.where(vmask, posv, 0)
            idv = plsc.load_gather(idsb, [posv])
            m = vmask & (idv >= chunk_lo) & (idv < chunk_lo + width)
            plsc.store_compressed(pp.at[pl.ds(pcnt, LANES)], posv, mask=m)
            plsc.store_compressed(pc.at[pl.ds(pcnt, LANES)],
                                  idv - chunk_lo, mask=m)
            pcnt = pcnt + plsc.all_reduce_population_count(m)[0]

            @pl.when(pcnt >= LANES)
            def _():
                emit(buf, width, jnp.int32(LANES), pp, pc, stag, g_hbm)
                rem_p = pp[pl.ds(LANES, LANES)]
                rem_c = pc[pl.ds(LANES, LANES)]
                pp[pl.ds(0, LANES)] = rem_p
                pc[pl.ds(0, LANES)] = rem_c

            return jnp.where(pcnt >= LANES, pcnt - LANES, pcnt)

        pcnt = lax.fori_loop(0, nvec, body, jnp.int32(0))

        @pl.when(pcnt > 0)
        def _():
            emit(buf, width, pcnt, pp, pc, stag, g_hbm)

    def process(buf, width, chunk_lo):
        process_side(buf, width, chunk_lo, cnt1, idsb1, posb1, pp1, pc1,
                     stag1, g1_hbm)
        process_side(buf, width, chunk_lo, cnt2, idsb2, posb2, pp2, pc2,
                     stag2, g2_hbm)

    fire(0, buf0, semb0)

    @pl.when(nch > 1)
    def _():
        fire(1, buf1, semb1)

    def chunk_pair(i, carry):
        q0 = 2 * i
        q1 = 2 * i + 1

        @pl.when(q0 < nch)
        def _():
            bwait(buf0, semb0)
            process(buf0, CHUNK, lo + q0 * CHUNK)

            @pl.when(q0 + 2 < nch)
            def _():
                fire(q0 + 2, buf0, semb0)

        @pl.when(q1 < nch)
        def _():
            bwait(buf1, semb1)
            process(buf1, CHUNK, lo + q1 * CHUNK)

            @pl.when(q1 + 2 < nch)
            def _():
                fire(q1 + 2, buf1, semb1)

        return carry

    lax.fori_loop(0, (CPW + 2) // 2, chunk_pair, 0)

    @pl.when(wid == NW - 1)
    def _():
        pltpu.sync_copy(tail_t_hbm, tbuf)
        process(tbuf, 64, jnp.int32(NUM_ROWS - 64))


def _dot_body(g1_ref, g2_ref, out_ref):
    out_ref[...] = jnp.sum(g1_ref[...] * g2_ref[...], axis=1, keepdims=True)


_TC_BLOCK = 2048


@jax.jit
def _row_dot(g1, g2):
    return pl.pallas_call(
        _dot_body,
        grid=(BATCH // _TC_BLOCK,),
        in_specs=[
            pl.BlockSpec((_TC_BLOCK, DIM), lambda i: (i, 0)),
            pl.BlockSpec((_TC_BLOCK, DIM), lambda i: (i, 0)),
        ],
        out_specs=pl.BlockSpec((_TC_BLOCK, 1), lambda i: (i, 0)),
        out_shape=jax.ShapeDtypeStruct((BATCH, 1), jnp.float32),
    )(g1, g2)


def kernel(all_gembs, ids_1, ids_2):
    g1, g2 = _gather_stream(all_gembs.T,
                            all_gembs[NUM_ROWS - 64:].T,
                            ids_1.astype(jnp.int32),
                            ids_2.astype(jnp.int32))
    return _row_dot(g1, g2)


# speedup vs baseline: 1.5936x; 1.5936x over previous
import functools

import jax
import jax.numpy as jnp
from jax import lax
from jax.experimental import pallas as pl
from jax.experimental.pallas import tpu as pltpu
from jax.experimental.pallas import tpu_sc as plsc

BATCH = 16384
DIM = 32
LANES = 16

_info = plsc.get_sparse_core_info()
NC = _info.num_cores
NS = _info.num_subcores
NW = NC * NS
NUM_ROWS = 1000000
CHUNK = 1024
POS_BITS = 14
POS_MASK = (1 << POS_BITS) - 1

_BOUNDS = [(w * (NUM_ROWS // NW)) // CHUNK * CHUNK for w in range(NW)]
_BOUNDS += [NUM_ROWS]
_MAXNCH = max((_BOUNDS[w + 1] - _BOUNDS[w]) // CHUNK for w in range(NW))
_W31_512 = _BOUNDS[31] + (_BOUNDS[32] - _BOUNDS[31]) // CHUNK * CHUNK

_mesh = plsc.VectorSubcoreMesh(core_axis_name="c", subcore_axis_name="s")


@functools.partial(
    pl.kernel,
    mesh=_mesh,
    compiler_params=pltpu.CompilerParams(
        needs_layout_passes=False, use_tc_tiling_on_sc=True),
    out_type=(jax.ShapeDtypeStruct((BATCH, DIM), jnp.float32),
              jax.ShapeDtypeStruct((BATCH, DIM), jnp.float32)),
    scratch_types=[
        pltpu.VMEM((BATCH,), jnp.int32),
        pltpu.VMEM((BATCH,), jnp.int32),
        pltpu.VMEM((DIM, CHUNK), jnp.float32),
        pltpu.VMEM((DIM, CHUNK), jnp.float32),
        pltpu.VMEM((DIM, 64), jnp.float32),
        pltpu.VMEM((2 * LANES,), jnp.int32),
        pltpu.VMEM((2 * LANES,), jnp.int32),
        pltpu.VMEM((LANES, DIM), jnp.float32),
        pltpu.VMEM((LANES, DIM), jnp.float32),
        pltpu.SemaphoreType.DMA,
        pltpu.SemaphoreType.DMA,
        pltpu.SemaphoreType.DMA,
    ],
)
def _gather_stream(table_t_hbm, tail_t_hbm, ids1_hbm, ids2_hbm, g1_hbm,
                   g2_hbm,
                   lst1, lst2, buf0, buf1, tbuf, pend1, pend2, stag1, stag2,
                   semb0, semb1, seme):
    wid = lax.axis_index("s") * NC + lax.axis_index("c")
    lo = (wid * (NUM_ROWS // NW)) // CHUNK * CHUNK
    hi_next = ((wid + 1) * (NUM_ROWS // NW)) // CHUNK * CHUNK
    hi = jnp.where(wid == NW - 1, NUM_ROWS, hi_next)
    nch = (hi_next - lo) // CHUNK

    lanesv = lax.iota(jnp.int32, LANES)

    pltpu.sync_copy(ids1_hbm, lst1)
    pltpu.sync_copy(ids2_hbm, lst2)

    def scan_body(i, cnts):
        c1, c2 = cnts
        pos = i * LANES + lanesv
        idv1 = lst1[pl.ds(i * LANES, LANES)]
        idv2 = lst2[pl.ds(i * LANES, LANES)]
        m1 = (idv1 >= lo) & (idv1 < hi)
        m2 = (idv2 >= lo) & (idv2 < hi)
        e1 = ((idv1 - lo) << POS_BITS) | pos
        e2 = ((idv2 - lo) << POS_BITS) | pos
        plsc.store_compressed(lst1.at[pl.ds(c1, LANES)], e1, mask=m1)
        plsc.store_compressed(lst2.at[pl.ds(c2, LANES)], e2, mask=m2)
        c1 = c1 + plsc.all_reduce_population_count(m1)[0]
        c2 = c2 + plsc.all_reduce_population_count(m2)[0]
        return c1, c2

    cnt1, cnt2 = lax.fori_loop(0, BATCH // LANES, scan_body,
                               (jnp.int32(0), jnp.int32(0)))

    def fire(q, buf, semb):
        off = pl.multiple_of(lo + q * CHUNK, 128)
        pltpu.async_copy(table_t_hbm.at[:, pl.ds(off, CHUNK)], buf, semb)

    def bwait(buf, semb):
        pltpu.make_async_copy(
            table_t_hbm.at[:, pl.ds(0, CHUNK)], buf, semb).wait()

    def emit(buf, k, qrel, pend, stag, g_hbm):
        ev = pend[pl.ds(0, LANES)]
        emask = lanesv < k
        colv = jnp.where(emask, (ev >> POS_BITS) - qrel, 0)
        posv = ev & POS_MASK
        for c in range(DIM):
            cc = jnp.full((LANES,), c, jnp.int32)
            vals = plsc.load_gather(buf, [cc, colv], mask=emask)
            plsc.store_scatter(stag, [lanesv, cc], vals, mask=emask)
        for s in range(LANES):
            @pl.when(s < k)
            def _():
                ps = posv[s]
                pltpu.async_copy(stag.at[pl.ds(s, 1)],
                                 g_hbm.at[pl.ds(ps, 1)], seme)
        def drain(s, carry):
            pltpu.make_async_copy(stag.at[pl.ds(0, 1)],
                                  g_hbm.at[pl.ds(0, 1)], seme).wait()
            return carry
        lax.fori_loop(0, k, drain, 0)

    def process_side(buf, width, qrel, cnt, lst, pend, stag, g_hbm):
        nvec = (cnt + LANES - 1) // LANES

        def body(j, pcnt):
            ev = lst[pl.ds(j * LANES, LANES)]
            rel = (ev >> POS_BITS) - qrel
            vmask = (j * LANES + lanesv) < cnt
            m = vmask & (rel >= 0) & (rel < width)
            plsc.store_compressed(pend.at[pl.ds(pcnt, LANES)], ev, mask=m)
            pcnt = pcnt + plsc.all_reduce_population_count(m)[0]

            @pl.when(pcnt >= LANES)
            def _():
                emit(buf, jnp.int32(LANES), qrel, pend, stag, g_hbm)
                pend[pl.ds(0, LANES)] = pend[pl.ds(LANES, LANES)]

            return jnp.where(pcnt >= LANES, pcnt - LANES, pcnt)

        pcnt = lax.fori_loop(0, nvec, body, jnp.int32(0))

        @pl.when(pcnt > 0)
        def _():
            emit(buf, pcnt, qrel, pend, stag, g_hbm)

    def process(buf, width, qrel):
        process_side(buf, width, qrel, cnt1, lst1, pend1, stag1, g1_hbm)
        process_side(buf, width, qrel, cnt2, lst2, pend2, stag2, g2_hbm)

    fire(0, buf0, semb0)

    @pl.when(nch > 1)
    def _():
        fire(1, buf1, semb1)

    def chunk_pair(i, carry):
        q0 = 2 * i
        q1 = 2 * i + 1

        @pl.when(q0 < nch)
        def _():
            bwait(buf0, semb0)
            process(buf0, CHUNK, q0 * CHUNK)

            @pl.when(q0 + 2 < nch)
            def _():
                fire(q0 + 2, buf0, semb0)

        @pl.when(q1 < nch)
        def _():
            bwait(buf1, semb1)
            process(buf1, CHUNK, q1 * CHUNK)

            @pl.when(q1 + 2 < nch)
            def _():
                fire(q1 + 2, buf1, semb1)

        return carry

    lax.fori_loop(0, (_MAXNCH + 1) // 2, chunk_pair, 0)

    @pl.when(wid == NW - 1)
    def _():
        off = pl.multiple_of(_W31_512, 128)
        pltpu.async_copy(
            table_t_hbm.at[:, pl.ds(off, 512)], buf0.at[:, pl.ds(0, 512)],
            semb0)
        pltpu.make_async_copy(
            table_t_hbm.at[:, pl.ds(off, 512)], buf0.at[:, pl.ds(0, 512)],
            semb0).wait()
        process(buf0, 512, jnp.int32(_W31_512) - lo)

        pltpu.sync_copy(tail_t_hbm, tbuf)
        process(tbuf, 64, jnp.int32(NUM_ROWS - 64) - lo)


def _dot_body(g1_ref, g2_ref, out_ref):
    out_ref[...] = jnp.sum(g1_ref[...] * g2_ref[...], axis=1, keepdims=True)


_TC_BLOCK = 2048


@jax.jit
def _row_dot(g1, g2):
    return pl.pallas_call(
        _dot_body,
        grid=(BATCH // _TC_BLOCK,),
        in_specs=[
            pl.BlockSpec((_TC_BLOCK, DIM), lambda i: (i, 0)),
            pl.BlockSpec((_TC_BLOCK, DIM), lambda i: (i, 0)),
        ],
        out_specs=pl.BlockSpec((_TC_BLOCK, 1), lambda i: (i, 0)),
        out_shape=jax.ShapeDtypeStruct((BATCH, 1), jnp.float32),
    )(g1, g2)


def kernel(all_gembs, ids_1, ids_2):
    g1, g2 = _gather_stream(all_gembs.T,
                            all_gembs[NUM_ROWS - 64:].T,
                            ids_1.astype(jnp.int32),
                            ids_2.astype(jnp.int32))
    return _row_dot(g1, g2)
